# Initial kernel scaffold; baseline (speedup 1.0000x reference)
import jax
import jax.numpy as jnp
from jax.experimental import pallas as pl


def kernel(inputs, labels, running_mean):
    raise NotImplementedError("write your pallas kernel here")


# speedup vs baseline: 2.3023x; 2.3023x over previous
import functools

import jax
import jax.numpy as jnp
from jax import lax
from jax.experimental import pallas as pl
from jax.experimental.pallas import tpu as pltpu
from jax.experimental.pallas import tpu_sc as plsc

_NUM_CLASSES = 10000
_D = 128
_N = 320000
_NC, _NS, _L = 2, 16, 16
_NW = _NC * _NS
_ROWS_PER_W = _N // _NW
_C = 80
_NCHUNK = _ROWS_PER_W // _C
_JREGS = _D // _L


@functools.partial(
    pl.kernel,
    out_type=jax.ShapeDtypeStruct((_NW, _L), jnp.float32),
    mesh=plsc.VectorSubcoreMesh(
        core_axis_name="c", subcore_axis_name="s",
        num_cores=_NC, num_subcores=_NS),
    scratch_types=[
        pltpu.VMEM((_C,), jnp.int32),
        pltpu.VMEM((_C, _D), jnp.float32),
        pltpu.VMEM((_C, _D), jnp.float32),
        pltpu.VMEM((_L,), jnp.float32),
        pltpu.SemaphoreType.DMA,
        pltpu.SemaphoreType.DMA,
    ],
)
def _sc_sqdist(x_hbm, lbl_hbm, tbl_hbm, out_hbm,
               idx_v, x_v, m_v, acc_v, sem_x, sem_m):
    wid = lax.axis_index("s") * _NC + lax.axis_index("c")
    base = wid * _ROWS_PER_W

    def chunk_body(ci, accs):
        row0 = base + ci * _C
        cp_x = pltpu.async_copy(x_hbm.at[pl.ds(row0, _C)], x_v, sem_x)
        pltpu.sync_copy(lbl_hbm.at[pl.ds(row0, _C)], idx_v)
        cp_m = pltpu.async_copy(tbl_hbm.at[idx_v], m_v, sem_m)
        cp_x.wait()
        cp_m.wait()

        def row_body(r, a):
            new = []
            for j in range(_JREGS):
                xv = x_v[r, pl.ds(j * _L, _L)]
                mv = m_v[r, pl.ds(j * _L, _L)]
                dv = xv - mv
                new.append(a[j] + dv * dv)
            return tuple(new)

        return lax.fori_loop(0, _C, row_body, accs)

    zero = jnp.zeros((_L,), jnp.float32)
    accs = lax.fori_loop(0, _NCHUNK, chunk_body, (zero,) * _JREGS)
    total = accs[0]
    for j in range(1, _JREGS):
        total = total + accs[j]
    acc_v[...] = total
    pltpu.sync_copy(acc_v, out_hbm.at[wid])


def kernel(inputs, labels, running_mean):
    partials = _sc_sqdist(inputs, labels.astype(jnp.int32), running_mean)
    regularization = jnp.sqrt(jnp.sum(partials))
    return inputs, regularization


# speedup vs baseline: 4.0234x; 1.7476x over previous
import functools

import jax
import jax.numpy as jnp
from jax import lax
from jax.experimental import pallas as pl
from jax.experimental.pallas import tpu as pltpu
from jax.experimental.pallas import tpu_sc as plsc

_NUM_CLASSES = 10000
_D = 128
_N = 320000
_NC, _NS, _L = 2, 16, 16
_NW = _NC * _NS
_ROWS_PER_W = _N // _NW
_C = 40
_NCHUNK = _ROWS_PER_W // _C
_NBUF = 5
_MAIN_T = _NCHUNK // _NBUF - 1
_JREGS = _D // _L


@functools.partial(
    pl.kernel,
    out_type=jax.ShapeDtypeStruct((_NW, _L), jnp.float32),
    mesh=plsc.VectorSubcoreMesh(
        core_axis_name="c", subcore_axis_name="s",
        num_cores=_NC, num_subcores=_NS),
    scratch_types=[
        pltpu.VMEM((_ROWS_PER_W,), jnp.int32),
        pltpu.VMEM((_NBUF, _C, _D), jnp.float32),
        pltpu.VMEM((_NBUF, _C, _D), jnp.float32),
        pltpu.VMEM((_L,), jnp.float32),
    ] + [pltpu.SemaphoreType.DMA] * (2 * _NBUF),
)
def _sc_sqdist(x_hbm, lbl_hbm, tbl_hbm, out_hbm,
               idx_all, x_v, m_v, acc_v, *sems):
    sem_x = sems[:_NBUF]
    sem_m = sems[_NBUF:]
    wid = lax.axis_index("s") * _NC + lax.axis_index("c")
    base = wid * _ROWS_PER_W

    pltpu.sync_copy(lbl_hbm.at[pl.ds(base, _ROWS_PER_W)], idx_all)

    def start(ci, b):
        row0 = base + ci * _C
        pltpu.async_copy(x_hbm.at[pl.ds(row0, _C)], x_v.at[b], sem_x[b])
        pltpu.async_copy(tbl_hbm.at[idx_all.at[pl.ds(ci * _C, _C)]],
                         m_v.at[b], sem_m[b])

    def wait(ci, b):
        row0 = base + ci * _C
        pltpu.make_async_copy(x_hbm.at[pl.ds(row0, _C)],
                              x_v.at[b], sem_x[b]).wait()
        pltpu.make_async_copy(tbl_hbm.at[pl.ds(0, _C)],
                              m_v.at[b], sem_m[b]).wait()

    def compute(b, accs):
        xb = x_v.at[b]
        mb = m_v.at[b]

        def row_body(r, a):
            new = []
            for j in range(_JREGS):
                dv = xb[r, pl.ds(j * _L, _L)] - mb[r, pl.ds(j * _L, _L)]
                new.append(a[j] + dv * dv)
            return tuple(new)

        return lax.fori_loop(0, _C, row_body, accs)

    for b in range(_NBUF):
        start(b, b)

    def ring_turn(t, accs):
        for b in range(_NBUF):
            ci = t * _NBUF + b
            wait(ci, b)
            accs = compute(b, accs)
            start(ci + _NBUF, b)
        return accs

    zero = jnp.zeros((_L,), jnp.float32)
    accs = lax.fori_loop(0, _MAIN_T, ring_turn, (zero,) * _JREGS)

    for b in range(_NBUF):
        ci = _MAIN_T * _NBUF + b
        wait(ci, b)
        accs = compute(b, accs)

    total = accs[0]
    for j in range(1, _JREGS):
        total = total + accs[j]
    acc_v[...] = total
    pltpu.sync_copy(acc_v, out_hbm.at[wid])


def kernel(inputs, labels, running_mean):
    partials = _sc_sqdist(inputs, labels.astype(jnp.int32), running_mean)
    regularization = jnp.sqrt(jnp.sum(partials))
    return inputs, regularization
